# baseline (speedup 1.0000x reference)
import jax
import jax.numpy as jnp
from jax import lax
from jax.experimental import pallas as pl
from jax.experimental.pallas import tpu as pltpu
from jax.experimental.pallas import tpu_sc as plsc

N = 10000
D = 128
H = 128
C = 40
EPS = 1e-5

NC = 2
NS = 16
NW = NC * NS
CHUNK = 128
N_PAD = 10240
ROWS_PER_TILE = N_PAD // NS
W1 = 144
W3 = 48


CPT = 80
NHALF = 2
NT_CHUNKS = NHALF * NS * CPT
NB = 2
NQ = 4


def _sc_mesh():
    return plsc.VectorSubcoreMesh(core_axis_name="c", subcore_axis_name="s")


def _make_sc_cnt():
    half = NT_CHUNKS // (NC * NS)

    def body(idx_hbm, cnt_out, dst_v, cnt_v):
        c = lax.axis_index("c")
        s = lax.axis_index("s")
        base = (c * NS + s) * half

        zeros16 = jnp.zeros((16,), jnp.float32)

        @pl.loop(0, N_PAD // 16)
        def _(i):
            cnt_v[pl.ds(i * 16, 16)] = zeros16

        pltpu.sync_copy(idx_hbm.at[pl.ds(base, half), 1], dst_v)

        ones16 = jnp.ones((16,), jnp.float32)

        @pl.loop(0, half)
        def _(j):
            for k in range(CHUNK // 16):
                idx = dst_v[j, pl.ds(k * 16, 16)]
                plsc.addupdate_scatter(cnt_v, [idx], ones16)

        pltpu.sync_copy(cnt_v, cnt_out.at[c, s])

    return pl.kernel(
        body,
        out_type=jax.ShapeDtypeStruct((NC, NS, N_PAD), jnp.float32),
        mesh=_sc_mesh(),
        compiler_params=pltpu.CompilerParams(use_tc_tiling_on_sc=False,
                                             needs_layout_passes=False),
        scratch_types=[
            pltpu.VMEM((half, CHUNK), jnp.int32),
            pltpu.VMEM((N_PAD,), jnp.float32),
        ],
    )


def _make_sc_agg(width, halfsel):
    def body(p_hbm, idx_hbm, acc_out, idx_v, rows_v, acc_sh, *sems):
        gsem = sems[:NB]
        ssem = sems[NB:2 * NB]
        isem = sems[2 * NB:]
        c = lax.axis_index("c")
        s = lax.axis_index("s")
        base = (halfsel * NS + s) * CPT

        @pl.when(c == 0)
        def _():
            zeros16 = jnp.zeros((16,), jnp.float32)

            @pl.loop(0, CHUNK)
            def _(r):
                for k in range(width // 16):
                    rows_v[0, r, pl.ds(k * 16, 16)] = zeros16

            row0 = s * ROWS_PER_TILE

            @pl.loop(0, ROWS_PER_TILE // CHUNK)
            def _(b):
                pltpu.sync_copy(rows_v.at[0],
                                acc_sh.at[pl.ds(row0 + b * CHUNK, CHUNK)])

            plsc.subcore_barrier()

            def ifetch(j, q):
                return pltpu.async_copy(idx_hbm.at[base + j], idx_v.at[q],
                                        isem[q])

            def iwait(j, q):
                pltpu.make_async_copy(idx_hbm.at[base + j], idx_v.at[q],
                                      isem[q]).wait()

            def gather_start(q, b):
                return pltpu.async_copy(p_hbm.at[idx_v.at[q, 0]],
                                        rows_v.at[b], gsem[b])

            def gather_wait(q, b):
                pltpu.make_async_copy(p_hbm.at[idx_v.at[q, 0]],
                                      rows_v.at[b], gsem[b]).wait()

            def scat_start(q, b):
                return pltpu.async_copy(rows_v.at[b],
                                        acc_sh.at[idx_v.at[q, 1]],
                                        ssem[b], add=True)

            def scat_wait(q, b):
                pltpu.make_async_copy(rows_v.at[b],
                                      acc_sh.at[idx_v.at[q, 1]],
                                      ssem[b]).wait()

            def step(t, b, first, last):
                j = t * NQ + b
                q = b % NQ
                if not (first and b == 0):
                    scat_wait((b - 1) % NQ, (b - 1) % NB)
                if not (last and b >= 3):
                    iwait(j + 1, (b + 1) % NQ)
                    gather_start((b + 1) % NQ, (b + 1) % NB)
                gather_wait(q, b % NB)
                scat_start(q, b % NB)
                if not (last and b >= 1):
                    ifetch(j + 3, (b + 3) % NQ)

            ifetch(0, 0)
            ifetch(1, 1)
            ifetch(2, 2)
            iwait(0, 0)
            gather_start(0, 0)

            for b in range(NQ):
                step(0, b, True, False)

            nt = CPT // NQ

            @pl.loop(1, nt - 1)
            def _(t):
                for b in range(NQ):
                    step(t, b, False, False)

            for b in range(NQ):
                step(nt - 1, b, False, True)
            scat_wait(3, (NQ - 1) % NB)

            plsc.subcore_barrier()

            rows = pl.ds(row0, ROWS_PER_TILE)
            pltpu.sync_copy(acc_sh.at[rows], acc_out.at[rows])

    return pl.kernel(
        body,
        out_type=jax.ShapeDtypeStruct((N_PAD, width), jnp.float32),
        mesh=_sc_mesh(),
        compiler_params=pltpu.CompilerParams(use_tc_tiling_on_sc=False),
        scratch_types=[
            pltpu.VMEM((NQ, 2, CHUNK), jnp.int32),
            pltpu.VMEM((NB, CHUNK, width), jnp.float32),
            pltpu.VMEM_SHARED((N_PAD, width), jnp.float32),
        ] + [pltpu.SemaphoreType.DMA] * (2 * NB + NQ),
    )


def _agg_pair(p, idx4, width):
    return (_make_sc_agg(width, 0)(p, idx4),
            _make_sc_agg(width, 1)(p, idx4))


def _tc_first(x, wl):
    def body(x_ref, w_ref, o_ref):
        o_ref[...] = jnp.dot(x_ref[...], w_ref[...],
                             preferred_element_type=jnp.float32)

    return pl.pallas_call(
        body,
        out_shape=jax.ShapeDtypeStruct((N, H), jnp.float32),
    )(x, wl)


def _tc_mid1(a0, a1, cnt, x, wr, b, g, be, wl_next):
    def body(a0_ref, a1_ref, cnt_ref, x_ref, wr_ref, b_ref, g_ref, be_ref,
             wln_ref, h_ref, p_ref, den_ref):
        csum = jnp.sum(cnt_ref[...], axis=(0, 1))
        denom = jnp.maximum(csum, 1.0)[:N, None]
        den_ref[...] = denom
        asum = a0_ref[:N, :] + a1_ref[:N, :]
        mean = asum / denom
        hpre = mean + jnp.dot(x_ref[...], wr_ref[...],
                              preferred_element_type=jnp.float32) + b_ref[...]
        mu = jnp.mean(hpre, axis=0)
        var = jnp.mean((hpre - mu) ** 2, axis=0)
        h = jnp.maximum((hpre - mu) / jnp.sqrt(var + EPS) * g_ref[...]
                        + be_ref[...], 0.0)
        h_ref[...] = h
        p_ref[...] = jnp.dot(h, wln_ref[...],
                             preferred_element_type=jnp.float32)

    return pl.pallas_call(
        body,
        out_shape=(
            jax.ShapeDtypeStruct((N, H), jnp.float32),
            jax.ShapeDtypeStruct((N, H), jnp.float32),
            jax.ShapeDtypeStruct((N, 1), jnp.float32),
        ),
    )(a0, a1, cnt, x, wr, b, g, be, wl_next)


def _tc_mid2(a0, a1, den, xin, wr, b, g, be, wl_next):
    def body(a0_ref, a1_ref, den_ref, x_ref, wr_ref, b_ref, g_ref, be_ref,
             wln_ref, h_ref, p_ref):
        asum = a0_ref[:N, :] + a1_ref[:N, :]
        mean = asum / den_ref[...]
        hpre = mean + jnp.dot(x_ref[...], wr_ref[...],
                              preferred_element_type=jnp.float32) + b_ref[...]
        mu = jnp.mean(hpre, axis=0)
        var = jnp.mean((hpre - mu) ** 2, axis=0)
        h = jnp.maximum((hpre - mu) / jnp.sqrt(var + EPS) * g_ref[...]
                        + be_ref[...], 0.0)
        h_ref[...] = h
        p_ref[...] = jnp.dot(h, wln_ref[...],
                             preferred_element_type=jnp.float32)

    return pl.pallas_call(
        body,
        out_shape=(
            jax.ShapeDtypeStruct((N, H), jnp.float32),
            jax.ShapeDtypeStruct((N, wl_next.shape[1]), jnp.float32),
        ),
    )(a0, a1, den, xin, wr, b, g, be, wl_next)


def _tc_final(a0, a1, den, xin, wr, b):
    def body(a0_ref, a1_ref, den_ref, x_ref, wr_ref, b_ref, o_ref):
        asum = a0_ref[:N, :C] + a1_ref[:N, :C]
        mean = asum / den_ref[...]
        o = mean + jnp.dot(x_ref[...], wr_ref[...],
                           preferred_element_type=jnp.float32) + b_ref[...]
        m = jnp.max(o, axis=1, keepdims=True)
        z = o - m
        lse = jnp.log(jnp.sum(jnp.exp(z), axis=1, keepdims=True))
        o_ref[...] = z - lse

    return pl.pallas_call(
        body,
        out_shape=jax.ShapeDtypeStruct((N, C), jnp.float32),
    )(a0, a1, den, xin, wr, b)


def kernel(x, edge_index, Wl1, Wr1, b1, g1, be1, Wl2, Wr2, b2, g2, be2,
           Wl3, Wr3, b3):
    e = edge_index.shape[1]
    e_pad = NT_CHUNKS * CHUNK

    src = edge_index[0].astype(jnp.int32)
    dst = edge_index[1].astype(jnp.int32)
    pad = e_pad - e
    if pad:
        src = jnp.concatenate([src, jnp.zeros((pad,), jnp.int32)])
        dst = jnp.concatenate([dst, jnp.full((pad,), N, jnp.int32)])
    idx4 = jnp.stack([src.reshape(NT_CHUNKS, CHUNK),
                      dst.reshape(NT_CHUNKS, CHUNK)], axis=1)

    wl3p = jnp.concatenate(
        [Wl3, jnp.zeros((H, W3 - C), jnp.float32)], axis=1)

    cnt = _make_sc_cnt()(idx4)
    p1 = _tc_first(x, Wl1)
    a10, a11 = _agg_pair(p1, idx4, H)
    h1, p2, den = _tc_mid1(a10, a11, cnt, x, Wr1, b1, g1, be1, Wl2)
    a20, a21 = _agg_pair(p2, idx4, H)
    h2, p3 = _tc_mid2(a20, a21, den, h1, Wr2, b2, g2, be2, wl3p)
    a30, a31 = _agg_pair(p3, idx4, W3)
    return _tc_final(a30, a31, den, h2, Wr3, b3)


# speedup vs baseline: 1.6052x; 1.6052x over previous
import jax
import jax.numpy as jnp
from jax import lax
from jax.experimental import pallas as pl
from jax.experimental.pallas import tpu as pltpu
from jax.experimental.pallas import tpu_sc as plsc

N = 10000
D = 128
H = 128
C = 40
EPS = 1e-5

NC = 2
NS = 16
NW = NC * NS
CHUNK = 128
N_PAD = 10240
ROWS_PER_TILE = N_PAD // NS
W1 = 144
W3 = 48


def _make_sc_agg(n_chunks, width):
    def body(p_hbm, src_hbm, dst_hbm, acc_out, src_v, dst_v, rows_v,
             acc_sh, sem):
        c = lax.axis_index("c")
        s = lax.axis_index("s")
        wid = c * NS + s

        zeros16 = jnp.zeros((16,), jnp.float32)

        @pl.loop(0, CHUNK)
        def _(r):
            for k in range(width // 16):
                rows_v[r, pl.ds(k * 16, 16)] = zeros16

        row0 = s * ROWS_PER_TILE

        @pl.loop(0, ROWS_PER_TILE // CHUNK)
        def _(b):
            pltpu.sync_copy(rows_v, acc_sh.at[pl.ds(row0 + b * CHUNK, CHUNK)])

        pltpu.sync_copy(src_hbm.at[wid], src_v)
        pltpu.sync_copy(dst_hbm.at[wid], dst_v)

        plsc.subcore_barrier()

        @pl.loop(0, n_chunks)
        def _(j):
            pltpu.async_copy(p_hbm.at[src_v.at[j]], rows_v, sem).wait()
            pltpu.sync_copy(rows_v, acc_sh.at[dst_v.at[j]], add=True)

        plsc.subcore_barrier()

        rows = pl.ds(row0, ROWS_PER_TILE)
        pltpu.sync_copy(acc_sh.at[rows], acc_out.at[c, rows])

    return pl.kernel(
        body,
        out_type=jax.ShapeDtypeStruct((NC, N_PAD, width), jnp.float32),
        mesh=plsc.VectorSubcoreMesh(core_axis_name="c", subcore_axis_name="s"),
        compiler_params=pltpu.CompilerParams(use_tc_tiling_on_sc=False),
        scratch_types=[
            pltpu.VMEM((n_chunks, CHUNK), jnp.int32),
            pltpu.VMEM((n_chunks, CHUNK), jnp.int32),
            pltpu.VMEM((CHUNK, width), jnp.float32),
            pltpu.VMEM_SHARED((N_PAD, width), jnp.float32),
            pltpu.SemaphoreType.DMA,
        ],
    )


def _tc_first(x, wl):
    def body(x_ref, w_ref, o_ref):
        o_ref[:, :H] = jnp.dot(x_ref[...], w_ref[...],
                               preferred_element_type=jnp.float32)
        col = lax.broadcasted_iota(jnp.int32, (N, W1 - H), 1)
        o_ref[:, H:] = jnp.where(col == 0, 1.0, 0.0)

    return pl.pallas_call(
        body,
        out_shape=jax.ShapeDtypeStruct((N, W1), jnp.float32),
    )(x, wl)


def _tc_mid1(agg, x, wr, b, g, be, wl_next):
    def body(agg_ref, x_ref, wr_ref, b_ref, g_ref, be_ref, wln_ref,
             h_ref, p_ref, den_ref):
        asum = agg_ref[0, :N, :] + agg_ref[1, :N, :]
        denom = jnp.maximum(asum[:, H:H + 1], 1.0)
        den_ref[...] = denom
        mean = asum[:, :H] / denom
        hpre = mean + jnp.dot(x_ref[...], wr_ref[...],
                              preferred_element_type=jnp.float32) + b_ref[...]
        mu = jnp.mean(hpre, axis=0)
        var = jnp.mean((hpre - mu) ** 2, axis=0)
        h = jnp.maximum((hpre - mu) / jnp.sqrt(var + EPS) * g_ref[...]
                        + be_ref[...], 0.0)
        h_ref[...] = h
        p_ref[...] = jnp.dot(h, wln_ref[...],
                             preferred_element_type=jnp.float32)

    return pl.pallas_call(
        body,
        out_shape=(
            jax.ShapeDtypeStruct((N, H), jnp.float32),
            jax.ShapeDtypeStruct((N, H), jnp.float32),
            jax.ShapeDtypeStruct((N, 1), jnp.float32),
        ),
    )(agg, x, wr, b, g, be, wl_next)


def _tc_mid2(agg, den, xin, wr, b, g, be, wl_next):
    def body(agg_ref, den_ref, x_ref, wr_ref, b_ref, g_ref, be_ref,
             wln_ref, h_ref, p_ref):
        asum = agg_ref[0, :N, :] + agg_ref[1, :N, :]
        mean = asum / den_ref[...]
        hpre = mean + jnp.dot(x_ref[...], wr_ref[...],
                              preferred_element_type=jnp.float32) + b_ref[...]
        mu = jnp.mean(hpre, axis=0)
        var = jnp.mean((hpre - mu) ** 2, axis=0)
        h = jnp.maximum((hpre - mu) / jnp.sqrt(var + EPS) * g_ref[...]
                        + be_ref[...], 0.0)
        h_ref[...] = h
        p_ref[...] = jnp.dot(h, wln_ref[...],
                             preferred_element_type=jnp.float32)

    return pl.pallas_call(
        body,
        out_shape=(
            jax.ShapeDtypeStruct((N, H), jnp.float32),
            jax.ShapeDtypeStruct((N, wl_next.shape[1]), jnp.float32),
        ),
    )(agg, den, xin, wr, b, g, be, wl_next)


def _tc_final(agg, den, xin, wr, b):
    def body(agg_ref, den_ref, x_ref, wr_ref, b_ref, o_ref):
        asum = agg_ref[0, :N, :C] + agg_ref[1, :N, :C]
        mean = asum / den_ref[...]
        o = mean + jnp.dot(x_ref[...], wr_ref[...],
                           preferred_element_type=jnp.float32) + b_ref[...]
        m = jnp.max(o, axis=1, keepdims=True)
        z = o - m
        lse = jnp.log(jnp.sum(jnp.exp(z), axis=1, keepdims=True))
        o_ref[...] = z - lse

    return pl.pallas_call(
        body,
        out_shape=jax.ShapeDtypeStruct((N, C), jnp.float32),
    )(agg, den, xin, wr, b)


def kernel(x, edge_index, Wl1, Wr1, b1, g1, be1, Wl2, Wr2, b2, g2, be2,
           Wl3, Wr3, b3):
    e = edge_index.shape[1]
    n_chunks = -(-e // (NW * CHUNK))
    e_pad = NW * n_chunks * CHUNK

    src = edge_index[0].astype(jnp.int32)
    dst = edge_index[1].astype(jnp.int32)
    pad = e_pad - e
    if pad:
        src = jnp.concatenate([src, jnp.zeros((pad,), jnp.int32)])
        dst = jnp.concatenate([dst, jnp.full((pad,), N, jnp.int32)])
    src3 = src.reshape(NW, n_chunks, CHUNK)
    dst3 = dst.reshape(NW, n_chunks, CHUNK)

    wl3p = jnp.concatenate(
        [Wl3, jnp.zeros((H, W3 - C), jnp.float32)], axis=1)

    p1 = _tc_first(x, Wl1)
    a1 = _make_sc_agg(n_chunks, W1)(p1, src3, dst3)
    h1, p2, den = _tc_mid1(a1, x, Wr1, b1, g1, be1, Wl2)
    a2 = _make_sc_agg(n_chunks, H)(p2, src3, dst3)
    h2, p3 = _tc_mid2(a2, den, h1, Wr2, b2, g2, be2, wl3p)
    a3 = _make_sc_agg(n_chunks, W3)(p3, src3, dst3)
    return _tc_final(a3, den, h2, Wr3, b3)
